# Initial kernel scaffold; baseline (speedup 1.0000x reference)
import jax
import jax.numpy as jnp
from jax.experimental import pallas as pl


def kernel(x, embed_weight):
    raise NotImplementedError("write your pallas kernel here")


# speedup vs baseline: 6.0887x; 6.0887x over previous
import functools

import jax
import jax.numpy as jnp
from jax import lax
from jax.experimental import pallas as pl
from jax.experimental.pallas import tpu as pltpu
from jax.experimental.pallas import tpu_sc as plsc

_VOCAB = 100000
_D = 64
_BATCH = 16384
_HIST = 50
_TOTAL = _BATCH * _HIST

_NC = 2
_NS = 16
_NW = _NC * _NS
_PER_W = _TOTAL // _NW
_CHUNK = 128
_NCH = _PER_W // _CHUNK
_NBUF = 4


def _emb_body(table, xr, out, idx_v, rows_v, *sems):
    gsems = sems[:_NBUF]
    ssems = sems[_NBUF:]
    c = lax.axis_index("c")
    s = lax.axis_index("s")
    wid = s * _NC + c
    base = wid * _PER_W

    pltpu.sync_copy(xr.at[wid], idx_v)

    def outer(it, carry):
        i = it * _NBUF
        gh = []
        for b in range(_NBUF):
            gh.append(
                pltpu.async_copy(table.at[idx_v.at[i + b]], rows_v.at[b], gsems[b])
            )
        sh = []
        for b in range(_NBUF):
            gh[b].wait()
            sh.append(
                pltpu.async_copy(
                    rows_v.at[b],
                    out.at[pl.ds(base + (i + b) * _CHUNK, _CHUNK)],
                    ssems[b],
                )
            )
        for b in range(_NBUF):
            sh[b].wait()
        return carry

    lax.fori_loop(0, _NCH // _NBUF, outer, 0)


@jax.jit
def _emb(x, embed_weight):
    xr = x.reshape(_NW, _NCH, _CHUNK)
    mesh = plsc.VectorSubcoreMesh(core_axis_name="c", subcore_axis_name="s")
    scratch = [
        pltpu.VMEM((_NCH, _CHUNK), jnp.int32),
        pltpu.VMEM((_NBUF, _CHUNK, _D), jnp.float32),
    ] + [pltpu.SemaphoreType.DMA] * (2 * _NBUF)
    out = pl.kernel(
        _emb_body,
        out_type=jax.ShapeDtypeStruct((_TOTAL, _D), jnp.float32),
        mesh=mesh,
        scratch_types=scratch,
        compiler_params=pltpu.CompilerParams(use_tc_tiling_on_sc=False),
    )(embed_weight, xr)
    return out.reshape(_BATCH, _HIST, _D)


def kernel(x, embed_weight):
    return _emb(x, embed_weight)


# speedup vs baseline: 6.2219x; 1.0219x over previous
import functools

import jax
import jax.numpy as jnp
from jax import lax
from jax.experimental import pallas as pl
from jax.experimental.pallas import tpu as pltpu
from jax.experimental.pallas import tpu_sc as plsc

_VOCAB = 100000
_D = 64
_BATCH = 16384
_HIST = 50
_TOTAL = _BATCH * _HIST

_NC = 2
_NS = 16
_NW = _NC * _NS
_PER_W = _TOTAL // _NW
_CHUNK = 128
_NCH = _PER_W // _CHUNK
_NBUF = 8


def _emb_body(table, xr, out, idx_v, rows_v, *sems):
    gsems = sems[:_NBUF]
    ssems = sems[_NBUF:]
    c = lax.axis_index("c")
    s = lax.axis_index("s")
    wid = s * _NC + c
    base = wid * _PER_W

    pltpu.sync_copy(xr.at[wid], idx_v)

    def fire_gather(i, b):
        return pltpu.async_copy(table.at[idx_v.at[i + b]], rows_v.at[b], gsems[b])

    def fire_store(i, b):
        return pltpu.async_copy(
            rows_v.at[b],
            out.at[pl.ds(base + (i + b) * _CHUNK, _CHUNK)],
            ssems[b],
        )

    def wait_store(b):
        pltpu.make_async_copy(
            rows_v.at[b], out.at[pl.ds(base, _CHUNK)], ssems[b]
        ).wait()

    gh = [fire_gather(0, b) for b in range(_NBUF)]
    for b in range(_NBUF):
        gh[b].wait()
        fire_store(0, b)

    def group(it, carry):
        i = it * _NBUF
        gh = []
        for b in range(_NBUF):
            wait_store(b)
            gh.append(fire_gather(i, b))
        for b in range(_NBUF):
            gh[b].wait()
            fire_store(i, b)
        return carry

    lax.fori_loop(1, _NCH // _NBUF, group, 0)

    for b in range(_NBUF):
        wait_store(b)


@jax.jit
def _emb(x, embed_weight):
    xr = x.reshape(_NW, _NCH, _CHUNK)
    mesh = plsc.VectorSubcoreMesh(core_axis_name="c", subcore_axis_name="s")
    scratch = [
        pltpu.VMEM((_NCH, _CHUNK), jnp.int32),
        pltpu.VMEM((_NBUF, _CHUNK, _D), jnp.float32),
    ] + [pltpu.SemaphoreType.DMA] * (2 * _NBUF)
    out = pl.kernel(
        _emb_body,
        out_type=jax.ShapeDtypeStruct((_TOTAL, _D), jnp.float32),
        mesh=mesh,
        scratch_types=scratch,
        compiler_params=pltpu.CompilerParams(use_tc_tiling_on_sc=False),
    )(embed_weight, xr)
    return out.reshape(_BATCH, _HIST, _D)


def kernel(x, embed_weight):
    return _emb(x, embed_weight)
